# Initial kernel scaffold; baseline (speedup 1.0000x reference)
import jax
import jax.numpy as jnp
from jax.experimental import pallas as pl


def kernel(x, importance, W_proj, b_proj, neuron_emb):
    raise NotImplementedError("write your pallas kernel here")


# speedup vs baseline: 1.1120x; 1.1120x over previous
import functools

import jax
import jax.numpy as jnp
from jax import lax
from jax.experimental import pallas as pl
from jax.experimental.pallas import tpu as pltpu

D_MODEL = 1024
N_GROUP = 64
K_C = 8
K_QK = 4
K_V = 6
S_CHUNK = 512


def _sparsify(row, k):
    sparse = jnp.zeros_like(row)
    work = row
    n = row.shape[-1]
    iota = lax.broadcasted_iota(jnp.int32, row.shape, row.ndim - 1)
    for _ in range(k):
        m = jnp.max(work, axis=-1, keepdims=True)
        eq = work == m
        min_idx = jnp.min(jnp.where(eq, iota, n), axis=-1, keepdims=True)
        first = iota == min_idx
        sparse = jnp.where(first, work, sparse)
        work = jnp.where(first, -jnp.inf, work)
    total = jnp.sum(sparse, axis=-1, keepdims=True)
    return sparse / (total + 1e-8)


def _body(x_ref, imp_ref, wt_ref, bp_ref, emb_ref,
          cw_ref, qw_ref, kw_ref, vw_ref,
          cacc, qkacc, vacc, *, nchunk):
    c = pl.program_id(1)

    xb = x_ref[0]
    h = jnp.dot(xb, wt_ref[...], preferred_element_type=jnp.float32)
    h = h + bp_ref[...]

    emb = emb_ref[...]
    norm = jnp.sqrt(jnp.sum(emb * emb, axis=-1, keepdims=True))
    emb_n = emb / jnp.maximum(norm, 1e-12)

    imp = imp_ref[0, 0, pl.ds(c * S_CHUNK, S_CHUNK)]
    impc = imp[:, None]

    def group_contrib(emb_g):
        lg = lax.dot_general(h, emb_g, (((1,), (1,)), ((), ())),
                             preferred_element_type=jnp.float32)
        m = jnp.max(lg, axis=-1, keepdims=True)
        e = jnp.exp(lg - m)
        p = e / jnp.sum(e, axis=-1, keepdims=True)
        return jnp.sum(impc * p, axis=0)[None, :]

    con_c = group_contrib(emb_n[0:64])
    con_qk = group_contrib(emb_n[64:128])
    con_v = group_contrib(emb_n[128:192])

    @pl.when(c == 0)
    def _init():
        cacc[...] = con_c
        qkacc[...] = con_qk
        vacc[...] = con_v

    @pl.when(c != 0)
    def _accum():
        cacc[...] += con_c
        qkacc[...] += con_qk
        vacc[...] += con_v

    @pl.when(c == nchunk - 1)
    def _finish():
        cw_ref[0] = _sparsify(cacc[...], K_C)
        qk = _sparsify(qkacc[...], K_QK)
        qw_ref[0] = qk
        kw_ref[0] = qk
        vw_ref[0] = _sparsify(vacc[...], K_V)


@jax.jit
def kernel(x, importance, W_proj, b_proj, neuron_emb):
    B, S, _ = x.shape
    nchunk = S // S_CHUNK
    wt = W_proj.T
    bp = b_proj[None, :]
    imp3 = importance[:, None, :]

    out_shape = [jax.ShapeDtypeStruct((B, 1, N_GROUP), jnp.float32)] * 4
    out_spec = pl.BlockSpec((1, 1, N_GROUP), lambda b, c: (b, 0, 0))

    outs = pl.pallas_call(
        functools.partial(_body, nchunk=nchunk),
        grid=(B, nchunk),
        in_specs=[
            pl.BlockSpec((1, S_CHUNK, D_MODEL), lambda b, c: (b, c, 0)),
            pl.BlockSpec((1, 1, S), lambda b, c: (b, 0, 0)),
            pl.BlockSpec(wt.shape, lambda b, c: (0, 0)),
            pl.BlockSpec(bp.shape, lambda b, c: (0, 0)),
            pl.BlockSpec(neuron_emb.shape, lambda b, c: (0, 0)),
        ],
        out_specs=[out_spec] * 4,
        out_shape=out_shape,
        scratch_shapes=[pltpu.VMEM((1, N_GROUP), jnp.float32)] * 3,
    )(x, imp3, wt, bp, neuron_emb)
    return tuple(o[:, 0, :] for o in outs)


# speedup vs baseline: 1.2137x; 1.0914x over previous
import functools

import jax
import jax.numpy as jnp
from jax import lax
from jax.experimental import pallas as pl
from jax.experimental.pallas import tpu as pltpu

D_MODEL = 1024
N_GROUP = 64
K_C = 8
K_QK = 4
K_V = 6
S_CHUNK = 512


def _sparsify(row, k):
    sparse = jnp.zeros_like(row)
    work = row
    n = row.shape[-1]
    iota = lax.broadcasted_iota(jnp.int32, row.shape, row.ndim - 1)
    for _ in range(k):
        m = jnp.max(work, axis=-1, keepdims=True)
        eq = work == m
        min_idx = jnp.min(jnp.where(eq, iota, n), axis=-1, keepdims=True)
        first = iota == min_idx
        sparse = jnp.where(first, work, sparse)
        work = jnp.where(first, -jnp.inf, work)
    total = jnp.sum(sparse, axis=-1, keepdims=True)
    return sparse / (total + 1e-8)


def _body(x_ref, imp_ref, w_ref, bp_ref, emb_ref,
          cw_ref, qw_ref, kw_ref, vw_ref,
          cacc, qkacc, vacc, *, nchunk):
    c = pl.program_id(1)

    xb = x_ref[0]
    ht = lax.dot_general(w_ref[...], xb, (((1,), (1,)), ((), ())),
                         preferred_element_type=jnp.float32)
    ht = ht + bp_ref[...]

    emb = emb_ref[...]
    norm = jnp.sqrt(jnp.sum(emb * emb, axis=-1, keepdims=True))
    emb_n = emb / jnp.maximum(norm, 1e-12)

    lt = lax.dot_general(emb_n, ht, (((1,), (0,)), ((), ())),
                         preferred_element_type=jnp.float32)

    imp_row = imp_ref[0, 0, pl.ds(c * S_CHUNK, S_CHUNK)][None, :]

    def group_contrib(lg):
        m = jnp.max(lg, axis=0, keepdims=True)
        e = jnp.exp(lg - m)
        s = jnp.sum(e, axis=0, keepdims=True)
        scale = imp_row / s
        return lax.dot_general(scale, e, (((1,), (1,)), ((), ())),
                               preferred_element_type=jnp.float32)

    con_c = group_contrib(lt[0:64])
    con_qk = group_contrib(lt[64:128])
    con_v = group_contrib(lt[128:192])

    @pl.when(c == 0)
    def _init():
        cacc[...] = con_c
        qkacc[...] = con_qk
        vacc[...] = con_v

    @pl.when(c != 0)
    def _accum():
        cacc[...] += con_c
        qkacc[...] += con_qk
        vacc[...] += con_v

    @pl.when(c == nchunk - 1)
    def _finish():
        cw_ref[0] = _sparsify(cacc[...], K_C)
        qk = _sparsify(qkacc[...], K_QK)
        qw_ref[0] = qk
        kw_ref[0] = qk
        vw_ref[0] = _sparsify(vacc[...], K_V)


@jax.jit
def kernel(x, importance, W_proj, b_proj, neuron_emb):
    B, S, _ = x.shape
    nchunk = S // S_CHUNK
    bp = b_proj[:, None]
    imp3 = importance[:, None, :]

    out_shape = [jax.ShapeDtypeStruct((B, 1, N_GROUP), jnp.float32)] * 4
    out_spec = pl.BlockSpec((1, 1, N_GROUP), lambda b, c: (b, 0, 0))

    outs = pl.pallas_call(
        functools.partial(_body, nchunk=nchunk),
        grid=(B, nchunk),
        in_specs=[
            pl.BlockSpec((1, S_CHUNK, D_MODEL), lambda b, c: (b, c, 0)),
            pl.BlockSpec((1, 1, S), lambda b, c: (b, 0, 0)),
            pl.BlockSpec(W_proj.shape, lambda b, c: (0, 0)),
            pl.BlockSpec(bp.shape, lambda b, c: (0, 0)),
            pl.BlockSpec(neuron_emb.shape, lambda b, c: (0, 0)),
        ],
        out_specs=[out_spec] * 4,
        out_shape=out_shape,
        scratch_shapes=[pltpu.VMEM((1, N_GROUP), jnp.float32)] * 3,
    )(x, imp3, W_proj, bp, neuron_emb)
    return tuple(o[:, 0, :] for o in outs)


# speedup vs baseline: 1.9575x; 1.6129x over previous
import functools

import jax
import jax.numpy as jnp
from jax import lax
from jax.experimental import pallas as pl
from jax.experimental.pallas import tpu as pltpu

D_MODEL = 1024
N_GROUP = 64
K_C = 8
K_QK = 4
K_V = 6
S_CHUNK = 512


def _sparsify(rows, k):
    sparse = jnp.zeros_like(rows)
    work = rows
    n = rows.shape[-1]
    iota = lax.broadcasted_iota(jnp.int32, rows.shape, rows.ndim - 1)
    for _ in range(k):
        m = jnp.max(work, axis=-1, keepdims=True)
        eq = work == m
        min_idx = jnp.min(jnp.where(eq, iota, n), axis=-1, keepdims=True)
        first = iota == min_idx
        sparse = jnp.where(first, work, sparse)
        work = jnp.where(first, -jnp.inf, work)
    total = jnp.sum(sparse, axis=-1, keepdims=True)
    return sparse / (total + 1e-8)


def _body(x_ref, mi_ref, w_ref, bp_ref, emb_ref,
          cw_ref, qw_ref, kw_ref, vw_ref,
          cacc, qkacc, vacc, *, nchunk, nb):
    c = pl.program_id(0)
    t = nb * S_CHUNK

    xf = x_ref[...].reshape(t, D_MODEL)
    ht = lax.dot_general(w_ref[...], xf, (((1,), (1,)), ((), ())),
                         preferred_element_type=jnp.float32)
    ht = ht + bp_ref[...]

    emb = emb_ref[...]
    norm = jnp.sqrt(jnp.sum(emb * emb, axis=-1, keepdims=True))
    emb_n = emb / jnp.maximum(norm, 1e-12)

    lt = lax.dot_general(emb_n, ht, (((1,), (0,)), ((), ())),
                         preferred_element_type=jnp.float32)

    mi = mi_ref[0]

    def group_contrib(lg):
        m = jnp.max(lg, axis=0, keepdims=True)
        e = jnp.exp(lg - m)
        s = jnp.sum(e, axis=0, keepdims=True)
        sw = mi * (1.0 / s)
        return lax.dot_general(sw, e, (((1,), (1,)), ((), ())),
                               preferred_element_type=jnp.float32)

    con_c = group_contrib(lt[0:64])
    con_qk = group_contrib(lt[64:128])
    con_v = group_contrib(lt[128:192])

    @pl.when(c == 0)
    def _init():
        cacc[...] = con_c
        qkacc[...] = con_qk
        vacc[...] = con_v

    @pl.when(c != 0)
    def _accum():
        cacc[...] += con_c
        qkacc[...] += con_qk
        vacc[...] += con_v

    @pl.when(c == nchunk - 1)
    def _finish():
        cw_ref[...] = _sparsify(cacc[...], K_C)
        qk = _sparsify(qkacc[...], K_QK)
        qw_ref[...] = qk
        kw_ref[...] = qk
        vw_ref[...] = _sparsify(vacc[...], K_V)


@jax.jit
def kernel(x, importance, W_proj, b_proj, neuron_emb):
    B, S, _ = x.shape
    nchunk = S // S_CHUNK
    t = B * S_CHUNK
    bp = b_proj[:, None]

    imp_chunks = importance.reshape(B, nchunk, S_CHUNK).transpose(1, 0, 2)
    eye = jnp.eye(B, dtype=importance.dtype)
    mi = (eye[None, :, :, None] * imp_chunks[:, None, :, :]).reshape(
        nchunk, B, t)

    out_shape = [jax.ShapeDtypeStruct((B, N_GROUP), jnp.float32)] * 4
    out_spec = pl.BlockSpec((B, N_GROUP), lambda c: (0, 0))

    outs = pl.pallas_call(
        functools.partial(_body, nchunk=nchunk, nb=B),
        grid=(nchunk,),
        in_specs=[
            pl.BlockSpec((B, S_CHUNK, D_MODEL), lambda c: (0, c, 0)),
            pl.BlockSpec((1, B, t), lambda c: (c, 0, 0)),
            pl.BlockSpec(W_proj.shape, lambda c: (0, 0)),
            pl.BlockSpec(bp.shape, lambda c: (0, 0)),
            pl.BlockSpec(neuron_emb.shape, lambda c: (0, 0)),
        ],
        out_specs=[out_spec] * 4,
        out_shape=out_shape,
        scratch_shapes=[pltpu.VMEM((B, N_GROUP), jnp.float32)] * 3,
    )(x, mi, W_proj, bp, neuron_emb)
    return tuple(outs)
